# Initial kernel scaffold; baseline (speedup 1.0000x reference)
import functools

import jax
import jax.numpy as jnp
from jax import lax
from jax.experimental import pallas as pl
from jax.experimental.pallas import tpu as pltpu
from jax.experimental.pallas import tpu_sc as plsc

N_NODES = 10000
N_P = 10240
IN_DIM = 128
HID_DIM = 128
OUT_DIM = 64

NC, NS = 2, 16
NW = NC * NS
E_P = 327680
E_W = E_P // NW
CHUNK = 128
N_CHUNKS = E_W // CHUNK
PAD_IDX = 10100

ROWS_PER_TILE = N_P // NS

_MESH = plsc.VectorSubcoreMesh(core_axis_name="c", subcore_axis_name="s")


def _deg_body(dst_hbm, out_hbm, didx, ones_v, acc_sh, zbuf):
    c = lax.axis_index("c")
    s = lax.axis_index("s")
    wid = c * NS + s

    zero16 = jnp.zeros((16,), jnp.float32)
    one16 = jnp.ones((16,), jnp.float32)
    for r in range(32):
        zbuf[r, :] = zero16
    for r in range(CHUNK):
        ones_v[r, :] = one16

    def _zero(j, _):
        pltpu.sync_copy(zbuf, acc_sh.at[pl.ds(s * ROWS_PER_TILE + j * 32, 32)])
        return 0
    lax.fori_loop(0, ROWS_PER_TILE // 32, _zero, 0)
    plsc.subcore_barrier()

    def _step(j, _):
        base = wid * E_W + j * CHUNK
        pltpu.sync_copy(dst_hbm.at[pl.ds(base, CHUNK)], didx.at[0])
        pltpu.sync_copy(ones_v, acc_sh.at[didx.at[0]], add=True)
        return 0
    lax.fori_loop(0, N_CHUNKS, _step, 0)
    plsc.subcore_barrier()

    pltpu.sync_copy(
        acc_sh.at[pl.ds(s * ROWS_PER_TILE, ROWS_PER_TILE)],
        out_hbm.at[c, pl.ds(s * ROWS_PER_TILE, ROWS_PER_TILE)],
    )


_deg_call = pl.kernel(
    _deg_body,
    out_type=jax.ShapeDtypeStruct((NC, N_P, 16), jnp.float32),
    mesh=_MESH,
    scratch_types=[
        pltpu.VMEM((1, CHUNK), jnp.int32),
        pltpu.VMEM((CHUNK, 16), jnp.float32),
        pltpu.VMEM_SHARED((N_P, 16), jnp.float32),
        pltpu.VMEM((32, 16), jnp.float32),
    ],
)


def _msg_body(xs_hbm, src_hbm, dst_hbm, out_hbm, sidx, didx, buf, acc_sh,
              zbuf, sem):
    c = lax.axis_index("c")
    s = lax.axis_index("s")
    wid = c * NS + s

    zero16 = jnp.zeros((16,), jnp.float32)
    for r in range(16):
        for l in range(8):
            zbuf[r, pl.ds(l * 16, 16)] = zero16

    def _zero(j, _):
        pltpu.sync_copy(zbuf, acc_sh.at[pl.ds(s * ROWS_PER_TILE + j * 16, 16)])
        return 0
    lax.fori_loop(0, ROWS_PER_TILE // 16, _zero, 0)
    plsc.subcore_barrier()

    def _step(j, _):
        base = wid * E_W + j * CHUNK
        pltpu.sync_copy(src_hbm.at[pl.ds(base, CHUNK)], sidx)
        pltpu.sync_copy(dst_hbm.at[pl.ds(base, CHUNK)], didx.at[0])
        pltpu.async_copy(xs_hbm.at[sidx], buf, sem).wait()
        pltpu.sync_copy(buf, acc_sh.at[didx.at[0]], add=True)
        return 0
    lax.fori_loop(0, N_CHUNKS, _step, 0)
    plsc.subcore_barrier()

    pltpu.sync_copy(
        acc_sh.at[pl.ds(s * ROWS_PER_TILE, ROWS_PER_TILE)],
        out_hbm.at[c, pl.ds(s * ROWS_PER_TILE, ROWS_PER_TILE)],
    )


_msg_call = pl.kernel(
    _msg_body,
    out_type=jax.ShapeDtypeStruct((NC, N_P, HID_DIM), jnp.float32),
    mesh=_MESH,
    scratch_types=[
        pltpu.VMEM((CHUNK,), jnp.int32),
        pltpu.VMEM((1, CHUNK), jnp.int32),
        pltpu.VMEM((CHUNK, HID_DIM), jnp.float32),
        pltpu.VMEM_SHARED((N_P, HID_DIM), jnp.float32),
        pltpu.VMEM((16, HID_DIM), jnp.float32),
        pltpu.SemaphoreType.DMA,
    ],
)


_R = 1280


def _scale_in_body(deg_ref, x_ref, w_ref, xs_ref, dinv_ref):
    deg = deg_ref[0, :, 0:1] + deg_ref[1, :, 0:1] + 1.0
    dinv = lax.rsqrt(deg)
    xw = jnp.dot(x_ref[...], w_ref[...], preferred_element_type=jnp.float32)
    xs_ref[...] = xw * dinv
    dinv_ref[...] = dinv


_scale_in_call = pl.pallas_call(
    _scale_in_body,
    grid=(N_P // _R,),
    in_specs=[
        pl.BlockSpec((NC, _R, 16), lambda i: (0, i, 0)),
        pl.BlockSpec((_R, IN_DIM), lambda i: (i, 0)),
        pl.BlockSpec((IN_DIM, HID_DIM), lambda i: (0, 0)),
    ],
    out_specs=[
        pl.BlockSpec((_R, HID_DIM), lambda i: (i, 0)),
        pl.BlockSpec((_R, 1), lambda i: (i, 0)),
    ],
    out_shape=[
        jax.ShapeDtypeStruct((N_P, HID_DIM), jnp.float32),
        jax.ShapeDtypeStruct((N_P, 1), jnp.float32),
    ],
)


def _mid_layer_body(s_ref, xs_ref, dinv_ref, b_ref, w_ref, out_ref):
    dinv = dinv_ref[...]
    h = (s_ref[0] + s_ref[1] + xs_ref[...]) * dinv + b_ref[...]
    h = jnp.maximum(h, 0.0)
    out_ref[...] = jnp.dot(
        h, w_ref[...], preferred_element_type=jnp.float32) * dinv


_mid_layer_call = pl.pallas_call(
    _mid_layer_body,
    grid=(N_P // _R,),
    in_specs=[
        pl.BlockSpec((NC, _R, HID_DIM), lambda i: (0, i, 0)),
        pl.BlockSpec((_R, HID_DIM), lambda i: (i, 0)),
        pl.BlockSpec((_R, 1), lambda i: (i, 0)),
        pl.BlockSpec((HID_DIM,), lambda i: (0,)),
        pl.BlockSpec((HID_DIM, HID_DIM), lambda i: (0, 0)),
    ],
    out_specs=pl.BlockSpec((_R, HID_DIM), lambda i: (i, 0)),
    out_shape=jax.ShapeDtypeStruct((N_P, HID_DIM), jnp.float32),
)


def _final_body(s_ref, xs_ref, dinv_ref, b_ref, wc_ref, bc_ref,
                out_ref, h_ref):
    h = (s_ref[0] + s_ref[1] + xs_ref[...]) * dinv_ref[...] + b_ref[...]
    h = jnp.maximum(h, 0.0)
    h_ref[...] = h
    out_ref[...] = jnp.dot(
        h, wc_ref[...], preferred_element_type=jnp.float32) + bc_ref[...]


_final_call = pl.pallas_call(
    _final_body,
    grid=(N_P // _R,),
    in_specs=[
        pl.BlockSpec((NC, _R, HID_DIM), lambda i: (0, i, 0)),
        pl.BlockSpec((_R, HID_DIM), lambda i: (i, 0)),
        pl.BlockSpec((_R, 1), lambda i: (i, 0)),
        pl.BlockSpec((HID_DIM,), lambda i: (0,)),
        pl.BlockSpec((HID_DIM, OUT_DIM), lambda i: (0, 0)),
        pl.BlockSpec((OUT_DIM,), lambda i: (0,)),
    ],
    out_specs=[
        pl.BlockSpec((_R, OUT_DIM), lambda i: (i, 0)),
        pl.BlockSpec((_R, HID_DIM), lambda i: (i, 0)),
    ],
    out_shape=[
        jax.ShapeDtypeStruct((N_P, OUT_DIM), jnp.float32),
        jax.ShapeDtypeStruct((N_P, HID_DIM), jnp.float32),
    ],
)


@jax.jit
def kernel(fts, edge_index, W1, b1, W2, b2, Wc, bc):
    n_edges = edge_index.shape[1]
    src = edge_index[0].astype(jnp.int32)
    dst = edge_index[1].astype(jnp.int32)
    pad = jnp.full((E_P - n_edges,), PAD_IDX, jnp.int32)
    src_p = jnp.concatenate([src, pad])
    dst_p = jnp.concatenate([dst, pad])
    fts_p = jnp.pad(fts, ((0, N_P - N_NODES), (0, 0)))

    deg_p = _deg_call(dst_p)
    xs1, dinv = _scale_in_call(deg_p, fts_p, W1)
    s1 = _msg_call(xs1, src_p, dst_p)
    xs2 = _mid_layer_call(s1, xs1, dinv, b1, W2)
    s2 = _msg_call(xs2, src_p, dst_p)
    out_p, h_p = _final_call(s2, xs2, dinv, b2, Wc, bc)
    return out_p[:N_NODES], h_p[:N_NODES]


# speedup vs baseline: 7.3515x; 7.3515x over previous
import functools

import jax
import jax.numpy as jnp
from jax import lax
from jax.experimental import pallas as pl
from jax.experimental.pallas import tpu as pltpu
from jax.experimental.pallas import tpu_sc as plsc

N_NODES = 10000
N_P = 10240
IN_DIM = 128
HID_DIM = 128
OUT_DIM = 64

NC, NS = 2, 16
NW = NC * NS
E_P = 327680
E_W = E_P // NW
CHUNK = 128
N_CHUNKS = E_W // CHUNK
PAD_IDX = 10100

ROWS_PER_TILE = N_P // NS

def _deg_body(dst_hbm, out_hbm, didx, ones_v, acc_sh, zbuf):
    c = lax.axis_index("c")
    s = lax.axis_index("s")
    wid = c * NS + s

    zero16 = jnp.zeros((16,), jnp.float32)
    one16 = jnp.ones((16,), jnp.float32)
    for r in range(32):
        zbuf[r, :] = zero16
    for r in range(CHUNK):
        ones_v[r, :] = one16

    def _zero(j, _):
        pltpu.sync_copy(zbuf, acc_sh.at[pl.ds(s * ROWS_PER_TILE + j * 32, 32)])
        return 0
    lax.fori_loop(0, ROWS_PER_TILE // 32, _zero, 0)
    plsc.subcore_barrier()

    def _step(j, _):
        base = wid * E_W + j * CHUNK
        pltpu.sync_copy(dst_hbm.at[pl.ds(base, CHUNK)], didx.at[0])
        pltpu.sync_copy(ones_v, acc_sh.at[didx.at[0]], add=True)
        return 0
    lax.fori_loop(0, N_CHUNKS, _step, 0)
    plsc.subcore_barrier()

    pltpu.sync_copy(
        acc_sh.at[pl.ds(s * ROWS_PER_TILE, ROWS_PER_TILE)],
        out_hbm.at[c, pl.ds(s * ROWS_PER_TILE, ROWS_PER_TILE)],
    )


@functools.cache
def _deg_call():
    return pl.kernel(
        _deg_body,
        out_type=jax.ShapeDtypeStruct((NC, N_P, 16), jnp.float32),
        mesh=plsc.VectorSubcoreMesh(
            core_axis_name="c", subcore_axis_name="s",
            num_cores=NC, num_subcores=NS),
        scratch_types=[
            pltpu.VMEM((1, CHUNK), jnp.int32),
            pltpu.VMEM((CHUNK, 16), jnp.float32),
            pltpu.VMEM_SHARED((N_P, 16), jnp.float32),
            pltpu.VMEM((32, 16), jnp.float32),
        ],
    )


def _msg_body(xs_hbm, src_hbm, dst_hbm, out_hbm, sidx, didx, buf, acc_sh,
              zbuf, sem):
    c = lax.axis_index("c")
    s = lax.axis_index("s")
    wid = c * NS + s

    zero16 = jnp.zeros((16,), jnp.float32)
    for r in range(16):
        for l in range(8):
            zbuf[r, pl.ds(l * 16, 16)] = zero16

    def _zero(j, _):
        pltpu.sync_copy(zbuf, acc_sh.at[pl.ds(s * ROWS_PER_TILE + j * 16, 16)])
        return 0
    lax.fori_loop(0, ROWS_PER_TILE // 16, _zero, 0)
    plsc.subcore_barrier()

    def _step(j, _):
        base = wid * E_W + j * CHUNK
        pltpu.sync_copy(src_hbm.at[pl.ds(base, CHUNK)], sidx)
        pltpu.sync_copy(dst_hbm.at[pl.ds(base, CHUNK)], didx.at[0])
        pltpu.async_copy(xs_hbm.at[sidx], buf, sem).wait()
        pltpu.sync_copy(buf, acc_sh.at[didx.at[0]], add=True)
        return 0
    lax.fori_loop(0, N_CHUNKS, _step, 0)
    plsc.subcore_barrier()

    pltpu.sync_copy(
        acc_sh.at[pl.ds(s * ROWS_PER_TILE, ROWS_PER_TILE)],
        out_hbm.at[c, pl.ds(s * ROWS_PER_TILE, ROWS_PER_TILE)],
    )


@functools.cache
def _msg_call():
    return pl.kernel(
        _msg_body,
        out_type=jax.ShapeDtypeStruct((NC, N_P, HID_DIM), jnp.float32),
        mesh=plsc.VectorSubcoreMesh(
            core_axis_name="c", subcore_axis_name="s",
            num_cores=NC, num_subcores=NS),
        scratch_types=[
            pltpu.VMEM((CHUNK,), jnp.int32),
            pltpu.VMEM((1, CHUNK), jnp.int32),
            pltpu.VMEM((CHUNK, HID_DIM), jnp.float32),
            pltpu.VMEM_SHARED((N_P, HID_DIM), jnp.float32),
            pltpu.VMEM((16, HID_DIM), jnp.float32),
            pltpu.SemaphoreType.DMA,
        ],
    )


_R = 1280


def _scale_in_body(deg_ref, x_ref, w_ref, xs_ref, dinv_ref):
    deg = deg_ref[0, :, 0:1] + deg_ref[1, :, 0:1] + 1.0
    dinv = lax.rsqrt(deg)
    xw = jnp.dot(x_ref[...], w_ref[...], preferred_element_type=jnp.float32)
    xs_ref[...] = xw * dinv
    dinv_ref[...] = dinv


_scale_in_call = pl.pallas_call(
    _scale_in_body,
    grid=(N_P // _R,),
    in_specs=[
        pl.BlockSpec((NC, _R, 16), lambda i: (0, i, 0)),
        pl.BlockSpec((_R, IN_DIM), lambda i: (i, 0)),
        pl.BlockSpec((IN_DIM, HID_DIM), lambda i: (0, 0)),
    ],
    out_specs=[
        pl.BlockSpec((_R, HID_DIM), lambda i: (i, 0)),
        pl.BlockSpec((_R, 1), lambda i: (i, 0)),
    ],
    out_shape=[
        jax.ShapeDtypeStruct((N_P, HID_DIM), jnp.float32),
        jax.ShapeDtypeStruct((N_P, 1), jnp.float32),
    ],
)


def _mid_layer_body(s_ref, xs_ref, dinv_ref, b_ref, w_ref, out_ref):
    dinv = dinv_ref[...]
    h = (s_ref[0] + s_ref[1] + xs_ref[...]) * dinv + b_ref[...]
    h = jnp.maximum(h, 0.0)
    out_ref[...] = jnp.dot(
        h, w_ref[...], preferred_element_type=jnp.float32) * dinv


_mid_layer_call = pl.pallas_call(
    _mid_layer_body,
    grid=(N_P // _R,),
    in_specs=[
        pl.BlockSpec((NC, _R, HID_DIM), lambda i: (0, i, 0)),
        pl.BlockSpec((_R, HID_DIM), lambda i: (i, 0)),
        pl.BlockSpec((_R, 1), lambda i: (i, 0)),
        pl.BlockSpec((HID_DIM,), lambda i: (0,)),
        pl.BlockSpec((HID_DIM, HID_DIM), lambda i: (0, 0)),
    ],
    out_specs=pl.BlockSpec((_R, HID_DIM), lambda i: (i, 0)),
    out_shape=jax.ShapeDtypeStruct((N_P, HID_DIM), jnp.float32),
)


def _final_body(s_ref, xs_ref, dinv_ref, b_ref, wc_ref, bc_ref,
                out_ref, h_ref):
    h = (s_ref[0] + s_ref[1] + xs_ref[...]) * dinv_ref[...] + b_ref[...]
    h = jnp.maximum(h, 0.0)
    h_ref[...] = h
    out_ref[...] = jnp.dot(
        h, wc_ref[...], preferred_element_type=jnp.float32) + bc_ref[...]


_final_call = pl.pallas_call(
    _final_body,
    grid=(N_P // _R,),
    in_specs=[
        pl.BlockSpec((NC, _R, HID_DIM), lambda i: (0, i, 0)),
        pl.BlockSpec((_R, HID_DIM), lambda i: (i, 0)),
        pl.BlockSpec((_R, 1), lambda i: (i, 0)),
        pl.BlockSpec((HID_DIM,), lambda i: (0,)),
        pl.BlockSpec((HID_DIM, OUT_DIM), lambda i: (0, 0)),
        pl.BlockSpec((OUT_DIM,), lambda i: (0,)),
    ],
    out_specs=[
        pl.BlockSpec((_R, OUT_DIM), lambda i: (i, 0)),
        pl.BlockSpec((_R, HID_DIM), lambda i: (i, 0)),
    ],
    out_shape=[
        jax.ShapeDtypeStruct((N_P, OUT_DIM), jnp.float32),
        jax.ShapeDtypeStruct((N_P, HID_DIM), jnp.float32),
    ],
)


@jax.jit
def kernel(fts, edge_index, W1, b1, W2, b2, Wc, bc):
    n_edges = edge_index.shape[1]
    src = edge_index[0].astype(jnp.int32)
    dst = edge_index[1].astype(jnp.int32)
    pad = jnp.full((E_P - n_edges,), PAD_IDX, jnp.int32)
    src_p = jnp.concatenate([src, pad])
    dst_p = jnp.concatenate([dst, pad])
    fts_p = jnp.pad(fts, ((0, N_P - N_NODES), (0, 0)))

    deg_p = _deg_call()(dst_p)
    xs1, dinv = _scale_in_call(deg_p, fts_p, W1)
    s1 = _msg_call()(xs1, src_p, dst_p)
    xs2 = _mid_layer_call(s1, xs1, dinv, b1, W2)
    s2 = _msg_call()(xs2, src_p, dst_p)
    out_p, h_p = _final_call(s2, xs2, dinv, b2, Wc, bc)
    return out_p[:N_NODES], h_p[:N_NODES]


# speedup vs baseline: 8.2315x; 1.1197x over previous
import functools

import jax
import jax.numpy as jnp
from jax import lax
from jax.experimental import pallas as pl
from jax.experimental.pallas import tpu as pltpu
from jax.experimental.pallas import tpu_sc as plsc

N_NODES = 10000
N_P = 10240
IN_DIM = 128
HID_DIM = 128
OUT_DIM = 64

NC, NS = 2, 16
NW = NC * NS
E_P = 327680
E_W = E_P // NW
CHUNK = 128
CPW = E_W // CHUNK
PAD_IDX = 10100

ROWS_PER_TILE = N_P // NS

_DNBUF = 4
_DNGRP = CPW // _DNBUF
_NBUF = 2
_NGRP = CPW // _NBUF


def _deg_body(dst_hbm, out_hbm, didx, ones_v, acc_sh, zbuf, *sems):
    jsem = sems[:_DNBUF]
    ssem = sems[_DNBUF:]
    c = lax.axis_index("c")
    s = lax.axis_index("s")
    wid = c * NS + s
    base = wid * CPW

    zero16 = jnp.zeros((16,), jnp.float32)
    one16 = jnp.ones((16,), jnp.float32)
    for r in range(32):
        zbuf[r, :] = zero16
    for r in range(CHUNK):
        ones_v[r, :] = one16

    def _didx_copy(j, b):
        return pltpu.make_async_copy(
            dst_hbm.at[pl.ds(base + j, 1)], didx.at[pl.ds(b, 1)], jsem[b])

    def _scatter(b):
        return pltpu.make_async_copy(ones_v, acc_sh.at[didx.at[b]], ssem[b])

    for b in range(_DNBUF):
        _didx_copy(b, b).start()

    def _zero(j, _):
        pltpu.sync_copy(zbuf, acc_sh.at[pl.ds(s * ROWS_PER_TILE + j * 32, 32)])
        return 0
    lax.fori_loop(0, ROWS_PER_TILE // 32, _zero, 0)
    plsc.subcore_barrier()

    def _grp(g, _):
        j0 = g * _DNBUF
        for b in range(_DNBUF):
            _didx_copy(j0 + b, b).wait()
            _scatter(b).start(add=True)
        for b in range(_DNBUF):
            _scatter(b).wait()

            @pl.when(g < _DNGRP - 1)
            def _():
                _didx_copy(j0 + _DNBUF + b, b).start()
        return 0
    lax.fori_loop(0, _DNGRP, _grp, 0)
    plsc.subcore_barrier()

    pltpu.sync_copy(
        acc_sh.at[pl.ds(s * ROWS_PER_TILE, ROWS_PER_TILE)],
        out_hbm.at[c, pl.ds(s * ROWS_PER_TILE, ROWS_PER_TILE)],
    )


@functools.cache
def _deg_call():
    return pl.kernel(
        _deg_body,
        out_type=jax.ShapeDtypeStruct((NC, N_P, 16), jnp.float32),
        mesh=plsc.VectorSubcoreMesh(
            core_axis_name="c", subcore_axis_name="s",
            num_cores=NC, num_subcores=NS),
        scratch_types=[
            pltpu.VMEM((_DNBUF, CHUNK), jnp.int32),
            pltpu.VMEM((CHUNK, 16), jnp.float32),
            pltpu.VMEM_SHARED((N_P, 16), jnp.float32),
            pltpu.VMEM((32, 16), jnp.float32),
        ] + [pltpu.SemaphoreType.DMA] * (2 * _DNBUF),
    )


def _msg_body(xs_hbm, src_hbm, dst_hbm, out_hbm, sidx, didx, bufs, acc_sh,
              zbuf, *sems):
    isem = sems[0 * _NBUF:1 * _NBUF]
    jsem = sems[1 * _NBUF:2 * _NBUF]
    gsem = sems[2 * _NBUF:3 * _NBUF]
    ssem = sems[3 * _NBUF:4 * _NBUF]
    c = lax.axis_index("c")
    s = lax.axis_index("s")
    wid = c * NS + s
    base = wid * CPW

    zero16 = jnp.zeros((16,), jnp.float32)
    for r in range(16):
        for l in range(8):
            zbuf[r, pl.ds(l * 16, 16)] = zero16

    def _sidx_copy(j, b):
        return pltpu.make_async_copy(
            src_hbm.at[pl.ds(base + j, 1)], sidx.at[pl.ds(b, 1)], isem[b])

    def _didx_copy(j, b):
        return pltpu.make_async_copy(
            dst_hbm.at[pl.ds(base + j, 1)], didx.at[pl.ds(b, 1)], jsem[b])

    def _gather(b):
        return pltpu.make_async_copy(xs_hbm.at[sidx.at[b]], bufs.at[b],
                                     gsem[b])

    def _scatter(b):
        return pltpu.make_async_copy(bufs.at[b], acc_sh.at[didx.at[b]],
                                     ssem[b])

    for b in range(_NBUF):
        _sidx_copy(b, b).start()
        _didx_copy(b, b).start()

    def _zero(j, _):
        pltpu.sync_copy(zbuf, acc_sh.at[pl.ds(s * ROWS_PER_TILE + j * 16, 16)])
        return 0
    lax.fori_loop(0, ROWS_PER_TILE // 16, _zero, 0)
    plsc.subcore_barrier()

    def _grp(g, _):
        j0 = g * _NBUF
        for b in range(_NBUF):
            _sidx_copy(j0 + b, b).wait()
            _gather(b).start()
        for b in range(_NBUF):
            _gather(b).wait()
            _didx_copy(j0 + b, b).wait()
            _scatter(b).start(add=True)
        for b in range(_NBUF):
            _scatter(b).wait()

            @pl.when(g < _NGRP - 1)
            def _():
                _sidx_copy(j0 + _NBUF + b, b).start()
                _didx_copy(j0 + _NBUF + b, b).start()
        return 0
    lax.fori_loop(0, _NGRP, _grp, 0)
    plsc.subcore_barrier()

    pltpu.sync_copy(
        acc_sh.at[pl.ds(s * ROWS_PER_TILE, ROWS_PER_TILE)],
        out_hbm.at[c, pl.ds(s * ROWS_PER_TILE, ROWS_PER_TILE)],
    )


@functools.cache
def _msg_call():
    return pl.kernel(
        _msg_body,
        out_type=jax.ShapeDtypeStruct((NC, N_P, HID_DIM), jnp.float32),
        mesh=plsc.VectorSubcoreMesh(
            core_axis_name="c", subcore_axis_name="s",
            num_cores=NC, num_subcores=NS),
        scratch_types=[
            pltpu.VMEM((_NBUF, CHUNK), jnp.int32),
            pltpu.VMEM((_NBUF, CHUNK), jnp.int32),
            pltpu.VMEM((_NBUF, CHUNK, HID_DIM), jnp.float32),
            pltpu.VMEM_SHARED((N_P, HID_DIM), jnp.float32),
            pltpu.VMEM((16, HID_DIM), jnp.float32),
        ] + [pltpu.SemaphoreType.DMA] * (4 * _NBUF),
    )


_R = 1280


def _scale_in_body(deg_ref, x_ref, w_ref, xs_ref, dinv_ref):
    deg = deg_ref[0, :, 0:1] + deg_ref[1, :, 0:1] + 1.0
    dinv = lax.rsqrt(deg)
    xw = jnp.dot(x_ref[...], w_ref[...], preferred_element_type=jnp.float32)
    xs_ref[...] = xw * dinv
    dinv_ref[...] = dinv


_scale_in_call = pl.pallas_call(
    _scale_in_body,
    grid=(N_P // _R,),
    in_specs=[
        pl.BlockSpec((NC, _R, 16), lambda i: (0, i, 0)),
        pl.BlockSpec((_R, IN_DIM), lambda i: (i, 0)),
        pl.BlockSpec((IN_DIM, HID_DIM), lambda i: (0, 0)),
    ],
    out_specs=[
        pl.BlockSpec((_R, HID_DIM), lambda i: (i, 0)),
        pl.BlockSpec((_R, 1), lambda i: (i, 0)),
    ],
    out_shape=[
        jax.ShapeDtypeStruct((N_P, HID_DIM), jnp.float32),
        jax.ShapeDtypeStruct((N_P, 1), jnp.float32),
    ],
)


def _mid_layer_body(s_ref, xs_ref, dinv_ref, b_ref, w_ref, out_ref):
    dinv = dinv_ref[...]
    h = (s_ref[0] + s_ref[1] + xs_ref[...]) * dinv + b_ref[...]
    h = jnp.maximum(h, 0.0)
    out_ref[...] = jnp.dot(
        h, w_ref[...], preferred_element_type=jnp.float32) * dinv


_mid_layer_call = pl.pallas_call(
    _mid_layer_body,
    grid=(N_P // _R,),
    in_specs=[
        pl.BlockSpec((NC, _R, HID_DIM), lambda i: (0, i, 0)),
        pl.BlockSpec((_R, HID_DIM), lambda i: (i, 0)),
        pl.BlockSpec((_R, 1), lambda i: (i, 0)),
        pl.BlockSpec((HID_DIM,), lambda i: (0,)),
        pl.BlockSpec((HID_DIM, HID_DIM), lambda i: (0, 0)),
    ],
    out_specs=pl.BlockSpec((_R, HID_DIM), lambda i: (i, 0)),
    out_shape=jax.ShapeDtypeStruct((N_P, HID_DIM), jnp.float32),
)


def _final_body(s_ref, xs_ref, dinv_ref, b_ref, wc_ref, bc_ref,
                out_ref, h_ref):
    h = (s_ref[0] + s_ref[1] + xs_ref[...]) * dinv_ref[...] + b_ref[...]
    h = jnp.maximum(h, 0.0)
    h_ref[...] = h
    out_ref[...] = jnp.dot(
        h, wc_ref[...], preferred_element_type=jnp.float32) + bc_ref[...]


_final_call = pl.pallas_call(
    _final_body,
    grid=(N_P // _R,),
    in_specs=[
        pl.BlockSpec((NC, _R, HID_DIM), lambda i: (0, i, 0)),
        pl.BlockSpec((_R, HID_DIM), lambda i: (i, 0)),
        pl.BlockSpec((_R, 1), lambda i: (i, 0)),
        pl.BlockSpec((HID_DIM,), lambda i: (0,)),
        pl.BlockSpec((HID_DIM, OUT_DIM), lambda i: (0, 0)),
        pl.BlockSpec((OUT_DIM,), lambda i: (0,)),
    ],
    out_specs=[
        pl.BlockSpec((_R, OUT_DIM), lambda i: (i, 0)),
        pl.BlockSpec((_R, HID_DIM), lambda i: (i, 0)),
    ],
    out_shape=[
        jax.ShapeDtypeStruct((N_P, OUT_DIM), jnp.float32),
        jax.ShapeDtypeStruct((N_P, HID_DIM), jnp.float32),
    ],
)


@jax.jit
def kernel(fts, edge_index, W1, b1, W2, b2, Wc, bc):
    n_edges = edge_index.shape[1]
    src = edge_index[0].astype(jnp.int32)
    dst = edge_index[1].astype(jnp.int32)
    pad = jnp.full((E_P - n_edges,), PAD_IDX, jnp.int32)
    src_p = jnp.concatenate([src, pad]).reshape(NW * CPW, CHUNK)
    dst_p = jnp.concatenate([dst, pad]).reshape(NW * CPW, CHUNK)
    fts_p = jnp.pad(fts, ((0, N_P - N_NODES), (0, 0)))

    deg_p = _deg_call()(dst_p)
    xs1, dinv = _scale_in_call(deg_p, fts_p, W1)
    s1 = _msg_call()(xs1, src_p, dst_p)
    xs2 = _mid_layer_call(s1, xs1, dinv, b1, W2)
    s2 = _msg_call()(xs2, src_p, dst_p)
    out_p, h_p = _final_call(s2, xs2, dinv, b2, Wc, bc)
    return out_p[:N_NODES], h_p[:N_NODES]


# speedup vs baseline: 23.4708x; 2.8513x over previous
import functools

import jax
import jax.numpy as jnp
from jax import lax
from jax.experimental import pallas as pl
from jax.experimental.pallas import tpu as pltpu
from jax.experimental.pallas import tpu_sc as plsc

N_NODES = 10000
N_P = 10240
IN_DIM = 128
HID_DIM = 128
OUT_DIM = 64

NC, NS = 2, 16
NW = NC * NS
E_P = 327680
E_W = E_P // NW
CHUNK = 128
CPW = E_W // CHUNK
PAD_IDX = 10100

ROWS_PER_TILE = N_P // NS

_DNBUF = 4
_DNGRP = CPW // _DNBUF
_NBUF = 2
_NGRP = CPW // _NBUF


def _deg_body(dst_hbm, out_hbm, didx, ones_v, acc_sh, zbuf, *sems):
    jsem = sems[:_DNBUF]
    ssem = sems[_DNBUF:]
    c = lax.axis_index("c")
    s = lax.axis_index("s")
    wid = c * NS + s
    base = wid * CPW

    zero16 = jnp.zeros((16,), jnp.float32)
    one16 = jnp.ones((16,), jnp.float32)
    for r in range(32):
        zbuf[r, :] = zero16
    for r in range(CHUNK):
        ones_v[r, :] = one16

    def _didx_copy(j, b):
        return pltpu.make_async_copy(
            dst_hbm.at[pl.ds(base + j, 1)], didx.at[pl.ds(b, 1)], jsem[b])

    def _scatter(b):
        return pltpu.make_async_copy(ones_v, acc_sh.at[didx.at[b]], ssem[b])

    for b in range(_DNBUF):
        _didx_copy(b, b).start()

    def _zero(j, _):
        pltpu.sync_copy(zbuf, acc_sh.at[pl.ds(s * ROWS_PER_TILE + j * 32, 32)])
        return 0
    lax.fori_loop(0, ROWS_PER_TILE // 32, _zero, 0)
    plsc.subcore_barrier()

    def _grp(g, _):
        j0 = g * _DNBUF
        for b in range(_DNBUF):
            _didx_copy(j0 + b, b).wait()
            _scatter(b).start(add=True)
        for b in range(_DNBUF):
            _scatter(b).wait()

            @pl.when(g < _DNGRP - 1)
            def _():
                _didx_copy(j0 + _DNBUF + b, b).start()
        return 0
    lax.fori_loop(0, _DNGRP, _grp, 0)
    plsc.subcore_barrier()

    pltpu.sync_copy(
        acc_sh.at[pl.ds(s * ROWS_PER_TILE, ROWS_PER_TILE)],
        out_hbm.at[c, pl.ds(s * ROWS_PER_TILE, ROWS_PER_TILE)],
    )


@functools.cache
def _deg_call():
    return pl.kernel(
        _deg_body,
        out_type=jax.ShapeDtypeStruct((NC, N_P, 16), jnp.float32),
        mesh=plsc.VectorSubcoreMesh(
            core_axis_name="c", subcore_axis_name="s",
            num_cores=NC, num_subcores=NS),
        scratch_types=[
            pltpu.VMEM((_DNBUF, CHUNK), jnp.int32),
            pltpu.VMEM((CHUNK, 16), jnp.float32),
            pltpu.VMEM_SHARED((N_P, 16), jnp.float32),
            pltpu.VMEM((32, 16), jnp.float32),
        ] + [pltpu.SemaphoreType.DMA] * (2 * _DNBUF),
    )


def _msg_body(xs_hbm, src_hbm, dst_hbm, out_hbm, sidx, didx, bufs, acc_sh,
              zbuf, *sems):
    isem = sems[0 * _NBUF:1 * _NBUF]
    jsem = sems[1 * _NBUF:2 * _NBUF]
    gsem = sems[2 * _NBUF:3 * _NBUF]
    ssem = sems[3 * _NBUF:4 * _NBUF]
    c = lax.axis_index("c")
    s = lax.axis_index("s")
    wid = c * NS + s
    base = wid * CPW

    zero16 = jnp.zeros((16,), jnp.float32)
    for r in range(16):
        for l in range(8):
            zbuf[r, pl.ds(l * 16, 16)] = zero16

    def _sidx_copy(j, b):
        return pltpu.make_async_copy(
            src_hbm.at[pl.ds(base + j, 1)], sidx.at[pl.ds(b, 1)], isem[b])

    def _didx_copy(j, b):
        return pltpu.make_async_copy(
            dst_hbm.at[pl.ds(base + j, 1)], didx.at[pl.ds(b, 1)], jsem[b])

    def _gather(b):
        return pltpu.make_async_copy(xs_hbm.at[sidx.at[b]], bufs.at[b],
                                     gsem[b])

    def _scatter(b):
        return pltpu.make_async_copy(bufs.at[b], acc_sh.at[didx.at[b]],
                                     ssem[b])

    for b in range(_NBUF):
        _sidx_copy(b, b).start()
        _didx_copy(b, b).start()

    def _zero(j, _):
        pltpu.sync_copy(zbuf, acc_sh.at[pl.ds(s * ROWS_PER_TILE + j * 16, 16)])
        return 0
    lax.fori_loop(0, ROWS_PER_TILE // 16, _zero, 0)
    plsc.subcore_barrier()

    def _grp(g, _):
        j0 = g * _NBUF
        for b in range(_NBUF):
            _sidx_copy(j0 + b, b).wait()
            _gather(b).start()
        for b in range(_NBUF):
            _gather(b).wait()
            _didx_copy(j0 + b, b).wait()
            _scatter(b).start(add=True)
        for b in range(_NBUF):
            _scatter(b).wait()

            @pl.when(g < _NGRP - 1)
            def _():
                _sidx_copy(j0 + _NBUF + b, b).start()
                _didx_copy(j0 + _NBUF + b, b).start()
        return 0
    lax.fori_loop(0, _NGRP, _grp, 0)
    plsc.subcore_barrier()

    pltpu.sync_copy(
        acc_sh.at[pl.ds(s * ROWS_PER_TILE, ROWS_PER_TILE)],
        out_hbm.at[c, pl.ds(s * ROWS_PER_TILE, ROWS_PER_TILE)],
    )


@functools.cache
def _msg_call():
    return pl.kernel(
        _msg_body,
        out_type=jax.ShapeDtypeStruct((NC, N_P, HID_DIM), jnp.float32),
        mesh=plsc.VectorSubcoreMesh(
            core_axis_name="c", subcore_axis_name="s",
            num_cores=NC, num_subcores=NS),
        scratch_types=[
            pltpu.VMEM((_NBUF, CHUNK), jnp.int32),
            pltpu.VMEM((_NBUF, CHUNK), jnp.int32),
            pltpu.VMEM((_NBUF, CHUNK, HID_DIM), jnp.float32),
            pltpu.VMEM_SHARED((N_P, HID_DIM), jnp.float32),
            pltpu.VMEM((16, HID_DIM), jnp.float32),
        ] + [pltpu.SemaphoreType.DMA] * (4 * _NBUF),
    )


_R = 1280


def _scale_in_body(deg_ref, x_ref, w_ref, xs_ref, dinv_ref):
    deg = deg_ref[0, :, 0:1] + deg_ref[1, :, 0:1] + 1.0
    dinv = lax.rsqrt(deg)
    xw = jnp.dot(x_ref[...], w_ref[...], preferred_element_type=jnp.float32)
    xs_ref[...] = xw * dinv
    dinv_ref[...] = dinv


_scale_in_call = pl.pallas_call(
    _scale_in_body,
    grid=(N_P // _R,),
    in_specs=[
        pl.BlockSpec((NC, _R, 16), lambda i: (0, i, 0)),
        pl.BlockSpec((_R, IN_DIM), lambda i: (i, 0)),
        pl.BlockSpec((IN_DIM, HID_DIM), lambda i: (0, 0)),
    ],
    out_specs=[
        pl.BlockSpec((_R, HID_DIM), lambda i: (i, 0)),
        pl.BlockSpec((_R, 1), lambda i: (i, 0)),
    ],
    out_shape=[
        jax.ShapeDtypeStruct((N_P, HID_DIM), jnp.float32),
        jax.ShapeDtypeStruct((N_P, 1), jnp.float32),
    ],
)


def _mid_layer_body(s_ref, xs_ref, dinv_ref, b_ref, w_ref, out_ref):
    dinv = dinv_ref[...]
    h = (s_ref[0] + s_ref[1] + xs_ref[...]) * dinv + b_ref[...]
    h = jnp.maximum(h, 0.0)
    out_ref[...] = jnp.dot(
        h, w_ref[...], preferred_element_type=jnp.float32) * dinv


_mid_layer_call = pl.pallas_call(
    _mid_layer_body,
    grid=(N_P // _R,),
    in_specs=[
        pl.BlockSpec((NC, _R, HID_DIM), lambda i: (0, i, 0)),
        pl.BlockSpec((_R, HID_DIM), lambda i: (i, 0)),
        pl.BlockSpec((_R, 1), lambda i: (i, 0)),
        pl.BlockSpec((HID_DIM,), lambda i: (0,)),
        pl.BlockSpec((HID_DIM, HID_DIM), lambda i: (0, 0)),
    ],
    out_specs=pl.BlockSpec((_R, HID_DIM), lambda i: (i, 0)),
    out_shape=jax.ShapeDtypeStruct((N_P, HID_DIM), jnp.float32),
)


def _final_body(s_ref, xs_ref, dinv_ref, b_ref, wc_ref, bc_ref,
                out_ref, h_ref):
    h = (s_ref[0] + s_ref[1] + xs_ref[...]) * dinv_ref[...] + b_ref[...]
    h = jnp.maximum(h, 0.0)
    h_ref[...] = h
    out_ref[...] = jnp.dot(
        h, wc_ref[...], preferred_element_type=jnp.float32) + bc_ref[...]


_final_call = pl.pallas_call(
    _final_body,
    grid=(N_P // _R,),
    in_specs=[
        pl.BlockSpec((NC, _R, HID_DIM), lambda i: (0, i, 0)),
        pl.BlockSpec((_R, HID_DIM), lambda i: (i, 0)),
        pl.BlockSpec((_R, 1), lambda i: (i, 0)),
        pl.BlockSpec((HID_DIM,), lambda i: (0,)),
        pl.BlockSpec((HID_DIM, OUT_DIM), lambda i: (0, 0)),
        pl.BlockSpec((OUT_DIM,), lambda i: (0,)),
    ],
    out_specs=[
        pl.BlockSpec((_R, OUT_DIM), lambda i: (i, 0)),
        pl.BlockSpec((_R, HID_DIM), lambda i: (i, 0)),
    ],
    out_shape=[
        jax.ShapeDtypeStruct((N_P, OUT_DIM), jnp.float32),
        jax.ShapeDtypeStruct((N_P, HID_DIM), jnp.float32),
    ],
)


@jax.jit
def kernel(fts, edge_index, W1, b1, W2, b2, Wc, bc):
    n_edges = edge_index.shape[1]
    src = edge_index[0].astype(jnp.int32)
    dst = edge_index[1].astype(jnp.int32)
    pad = N_NODES + (jnp.arange(E_P - n_edges, dtype=jnp.int32)
                     % (N_P - N_NODES))
    src_p = jnp.concatenate([src, pad]).reshape(NW * CPW, CHUNK)
    dst_p = jnp.concatenate([dst, pad]).reshape(NW * CPW, CHUNK)
    fts_p = jnp.pad(fts, ((0, N_P - N_NODES), (0, 0)))

    deg_p = _deg_call()(dst_p)
    xs1, dinv = _scale_in_call(deg_p, fts_p, W1)
    s1 = _msg_call()(xs1, src_p, dst_p)
    xs2 = _mid_layer_call(s1, xs1, dinv, b1, W2)
    s2 = _msg_call()(xs2, src_p, dst_p)
    out_p, h_p = _final_call(s2, xs2, dinv, b2, Wc, bc)
    return out_p[:N_NODES], h_p[:N_NODES]


# speedup vs baseline: 27.5343x; 1.1731x over previous
import functools

import jax
import jax.numpy as jnp
from jax import lax
from jax.experimental import pallas as pl
from jax.experimental.pallas import tpu as pltpu
from jax.experimental.pallas import tpu_sc as plsc

N_NODES = 10000
N_P = 10240
IN_DIM = 128
HID_DIM = 128
OUT_DIM = 64

NC, NS = 2, 16
NW = NC * NS
E_P = 327680
E_W = E_P // NW
CHUNK = 128
CPW = E_W // CHUNK
PAD_IDX = 10100

ROWS_PER_TILE = N_P // NS

_DNBUF = 4
_DNGRP = CPW // _DNBUF
_NBUF = 2
_NIDX = 4


def _deg_body(dst_hbm, out_hbm, didx, ones_v, acc_sh, zbuf, *sems):
    jsem = sems[:_DNBUF]
    ssem = sems[_DNBUF:]
    c = lax.axis_index("c")
    s = lax.axis_index("s")
    wid = c * NS + s
    base = wid * CPW

    zero16 = jnp.zeros((16,), jnp.float32)
    one16 = jnp.ones((16,), jnp.float32)
    for r in range(32):
        zbuf[r, :] = zero16
    for r in range(CHUNK):
        ones_v[r, :] = one16

    def _didx_copy(j, b):
        return pltpu.make_async_copy(
            dst_hbm.at[pl.ds(base + j, 1)], didx.at[pl.ds(b, 1)], jsem[b])

    def _scatter(b):
        return pltpu.make_async_copy(ones_v, acc_sh.at[didx.at[b]], ssem[b])

    for b in range(_DNBUF):
        _didx_copy(b, b).start()

    def _zero(j, _):
        pltpu.sync_copy(zbuf, acc_sh.at[pl.ds(s * ROWS_PER_TILE + j * 32, 32)])
        return 0
    lax.fori_loop(0, ROWS_PER_TILE // 32, _zero, 0)
    plsc.subcore_barrier()

    def _grp(g, _):
        j0 = g * _DNBUF
        for b in range(_DNBUF):
            _didx_copy(j0 + b, b).wait()
            _scatter(b).start(add=True)
        for b in range(_DNBUF):
            _scatter(b).wait()

            @pl.when(g < _DNGRP - 1)
            def _():
                _didx_copy(j0 + _DNBUF + b, b).start()
        return 0
    lax.fori_loop(0, _DNGRP, _grp, 0)
    plsc.subcore_barrier()

    pltpu.sync_copy(
        acc_sh.at[pl.ds(s * ROWS_PER_TILE, ROWS_PER_TILE)],
        out_hbm.at[c, pl.ds(s * ROWS_PER_TILE, ROWS_PER_TILE)],
    )


@functools.cache
def _deg_call():
    return pl.kernel(
        _deg_body,
        out_type=jax.ShapeDtypeStruct((NC, N_P, 16), jnp.float32),
        mesh=plsc.VectorSubcoreMesh(
            core_axis_name="c", subcore_axis_name="s",
            num_cores=NC, num_subcores=NS),
        scratch_types=[
            pltpu.VMEM((_DNBUF, CHUNK), jnp.int32),
            pltpu.VMEM((CHUNK, 16), jnp.float32),
            pltpu.VMEM_SHARED((N_P, 16), jnp.float32),
            pltpu.VMEM((32, 16), jnp.float32),
        ] + [pltpu.SemaphoreType.DMA] * (2 * _DNBUF),
    )


def _msg_body(xs_hbm, src_hbm, dst_hbm, out_hbm, sidx, didx, bufs, acc_sh,
              zbuf, *sems):
    isem = sems[0:_NIDX]
    jsem = sems[_NIDX:2 * _NIDX]
    gsem = sems[2 * _NIDX:2 * _NIDX + _NBUF]
    ssem = sems[2 * _NIDX + _NBUF:]
    c = lax.axis_index("c")
    s = lax.axis_index("s")
    wid = c * NS + s
    base = wid * CPW

    zero16 = jnp.zeros((16,), jnp.float32)
    for r in range(16):
        for l in range(8):
            zbuf[r, pl.ds(l * 16, 16)] = zero16

    def _idx_copy(j, i):
        return (pltpu.make_async_copy(
                    src_hbm.at[pl.ds(base + j, 1)],
                    sidx.at[pl.ds(i, 1)], isem[i]),
                pltpu.make_async_copy(
                    dst_hbm.at[pl.ds(base + j, 1)],
                    didx.at[pl.ds(i, 1)], jsem[i]))

    def _idx_start(j, i):
        a, d = _idx_copy(j, i)
        a.start()
        d.start()

    def _idx_wait(j, i):
        a, d = _idx_copy(j, i)
        a.wait()
        d.wait()

    def _gather(i, b):
        return pltpu.make_async_copy(xs_hbm.at[sidx.at[i]], bufs.at[b],
                                     gsem[b])

    def _scatter(i, b):
        return pltpu.make_async_copy(bufs.at[b], acc_sh.at[didx.at[i]],
                                     ssem[b])

    _idx_start(0, 0)
    _idx_start(1, 1)

    def _zero(j, _):
        pltpu.sync_copy(zbuf, acc_sh.at[pl.ds(s * ROWS_PER_TILE + j * 16, 16)])
        return 0
    lax.fori_loop(0, ROWS_PER_TILE // 16, _zero, 0)
    plsc.subcore_barrier()

    _idx_wait(0, 0)
    _gather(0, 0).start()

    def _sup(u, _):
        for q in range(_NIDX):
            j = _NIDX * u + q
            b = q % 2
            ob = 1 - b
            i1 = (q + 1) % _NIDX
            i2 = (q + 2) % _NIDX
            ip = (q + 3) % _NIDX
            _gather(q, b).wait()
            _scatter(q, b).start(add=True)

            @pl.when(j > 0)
            def _():
                _scatter(ip, ob).wait()

            @pl.when(j < CPW - 2)
            def _():
                _idx_start(j + 2, i2)

            @pl.when(j < CPW - 1)
            def _():
                _idx_wait(j + 1, i1)
                _gather(i1, ob).start()
        return 0
    lax.fori_loop(0, CPW // _NIDX, _sup, 0)
    _scatter((CPW - 1) % 4, 1).wait()
    plsc.subcore_barrier()

    pltpu.sync_copy(
        acc_sh.at[pl.ds(s * ROWS_PER_TILE, ROWS_PER_TILE)],
        out_hbm.at[c, pl.ds(s * ROWS_PER_TILE, ROWS_PER_TILE)],
    )


@functools.cache
def _msg_call():
    return pl.kernel(
        _msg_body,
        out_type=jax.ShapeDtypeStruct((NC, N_P, HID_DIM), jnp.float32),
        mesh=plsc.VectorSubcoreMesh(
            core_axis_name="c", subcore_axis_name="s",
            num_cores=NC, num_subcores=NS),
        scratch_types=[
            pltpu.VMEM((_NIDX, CHUNK), jnp.int32),
            pltpu.VMEM((_NIDX, CHUNK), jnp.int32),
            pltpu.VMEM((_NBUF, CHUNK, HID_DIM), jnp.float32),
            pltpu.VMEM_SHARED((N_P, HID_DIM), jnp.float32),
            pltpu.VMEM((16, HID_DIM), jnp.float32),
        ] + [pltpu.SemaphoreType.DMA] * (2 * _NIDX + 2 * _NBUF),
    )


_R = 1280


def _scale_in_body(deg_ref, x_ref, w_ref, xs_ref, dinv_ref):
    deg = deg_ref[0, :, 0:1] + deg_ref[1, :, 0:1] + 1.0
    dinv = lax.rsqrt(deg)
    xw = jnp.dot(x_ref[...], w_ref[...], preferred_element_type=jnp.float32)
    xs_ref[...] = xw * dinv
    dinv_ref[...] = dinv


_scale_in_call = pl.pallas_call(
    _scale_in_body,
    grid=(N_P // _R,),
    in_specs=[
        pl.BlockSpec((NC, _R, 16), lambda i: (0, i, 0)),
        pl.BlockSpec((_R, IN_DIM), lambda i: (i, 0)),
        pl.BlockSpec((IN_DIM, HID_DIM), lambda i: (0, 0)),
    ],
    out_specs=[
        pl.BlockSpec((_R, HID_DIM), lambda i: (i, 0)),
        pl.BlockSpec((_R, 1), lambda i: (i, 0)),
    ],
    out_shape=[
        jax.ShapeDtypeStruct((N_P, HID_DIM), jnp.float32),
        jax.ShapeDtypeStruct((N_P, 1), jnp.float32),
    ],
)


def _mid_layer_body(s_ref, xs_ref, dinv_ref, b_ref, w_ref, out_ref):
    dinv = dinv_ref[...]
    h = (s_ref[0] + s_ref[1] + xs_ref[...]) * dinv + b_ref[...]
    h = jnp.maximum(h, 0.0)
    out_ref[...] = jnp.dot(
        h, w_ref[...], preferred_element_type=jnp.float32) * dinv


_mid_layer_call = pl.pallas_call(
    _mid_layer_body,
    grid=(N_P // _R,),
    in_specs=[
        pl.BlockSpec((NC, _R, HID_DIM), lambda i: (0, i, 0)),
        pl.BlockSpec((_R, HID_DIM), lambda i: (i, 0)),
        pl.BlockSpec((_R, 1), lambda i: (i, 0)),
        pl.BlockSpec((HID_DIM,), lambda i: (0,)),
        pl.BlockSpec((HID_DIM, HID_DIM), lambda i: (0, 0)),
    ],
    out_specs=pl.BlockSpec((_R, HID_DIM), lambda i: (i, 0)),
    out_shape=jax.ShapeDtypeStruct((N_P, HID_DIM), jnp.float32),
)


def _final_body(s_ref, xs_ref, dinv_ref, b_ref, wc_ref, bc_ref,
                out_ref, h_ref):
    h = (s_ref[0] + s_ref[1] + xs_ref[...]) * dinv_ref[...] + b_ref[...]
    h = jnp.maximum(h, 0.0)
    h_ref[...] = h
    out_ref[...] = jnp.dot(
        h, wc_ref[...], preferred_element_type=jnp.float32) + bc_ref[...]


_final_call = pl.pallas_call(
    _final_body,
    grid=(N_P // _R,),
    in_specs=[
        pl.BlockSpec((NC, _R, HID_DIM), lambda i: (0, i, 0)),
        pl.BlockSpec((_R, HID_DIM), lambda i: (i, 0)),
        pl.BlockSpec((_R, 1), lambda i: (i, 0)),
        pl.BlockSpec((HID_DIM,), lambda i: (0,)),
        pl.BlockSpec((HID_DIM, OUT_DIM), lambda i: (0, 0)),
        pl.BlockSpec((OUT_DIM,), lambda i: (0,)),
    ],
    out_specs=[
        pl.BlockSpec((_R, OUT_DIM), lambda i: (i, 0)),
        pl.BlockSpec((_R, HID_DIM), lambda i: (i, 0)),
    ],
    out_shape=[
        jax.ShapeDtypeStruct((N_P, OUT_DIM), jnp.float32),
        jax.ShapeDtypeStruct((N_P, HID_DIM), jnp.float32),
    ],
)


@jax.jit
def kernel(fts, edge_index, W1, b1, W2, b2, Wc, bc):
    n_edges = edge_index.shape[1]
    src = edge_index[0].astype(jnp.int32)
    dst = edge_index[1].astype(jnp.int32)
    pad = N_NODES + (jnp.arange(E_P - n_edges, dtype=jnp.int32)
                     % (N_P - N_NODES))
    src_p = jnp.concatenate([src, pad]).reshape(NW * CPW, CHUNK)
    dst_p = jnp.concatenate([dst, pad]).reshape(NW * CPW, CHUNK)
    fts_p = jnp.pad(fts, ((0, N_P - N_NODES), (0, 0)))

    deg_p = _deg_call()(dst_p)
    xs1, dinv = _scale_in_call(deg_p, fts_p, W1)
    s1 = _msg_call()(xs1, src_p, dst_p)
    xs2 = _mid_layer_call(s1, xs1, dinv, b1, W2)
    s2 = _msg_call()(xs2, src_p, dst_p)
    out_p, h_p = _final_call(s2, xs2, dinv, b2, Wc, bc)
    return out_p[:N_NODES], h_p[:N_NODES]
